# Initial kernel scaffold; baseline (speedup 1.0000x reference)
import jax
import jax.numpy as jnp
from jax.experimental import pallas as pl


def kernel(x, edge_attr, params, edge_index, batch):
    raise NotImplementedError("write your pallas kernel here")


# speedup vs baseline: 1.0001x; 1.0001x over previous
import jax, jax.numpy as jnp
from jax.experimental import pallas as pl

T_SNAP = 4
N_GRAPHS = 16
HEADS = 4
HID = 64
LSTM_H = 128


def _bn(x, g, b):
    m = x.mean(axis=0)
    v = x.var(axis=0)
    return (x - m) / jnp.sqrt(v + 1e-5) * g + b


def _seg_softmax(a, seg, n):
    m = jax.ops.segment_max(a, seg, num_segments=n)
    ex = jnp.exp(a - m[seg])
    s = jax.ops.segment_sum(ex, seg, num_segments=n)
    return ex / (s[seg] + 1e-16)


def _tconv(x, ei, ea, p, heads, oc, concat):
    n = x.shape[0]
    src, dst = ei[0], ei[1]
    q = (x @ p['Wq'] + p['bq']).reshape(n, heads, oc)
    k = (x @ p['Wk'] + p['bk']).reshape(n, heads, oc)
    v = (x @ p['Wv'] + p['bv']).reshape(n, heads, oc)
    e = (ea @ p['We'] + p['be']).reshape(-1, heads, oc)
    kj = k[src] + e
    vj = v[src] + e
    qi = q[dst]
    alpha = (qi * kj).sum(-1) / jnp.sqrt(float(oc))
    alpha = _seg_softmax(alpha, dst, n)
    out = jax.ops.segment_sum(alpha[:, :, None] * vj, dst, num_segments=n)
    out = out.reshape(n, heads * oc) if concat else out.mean(axis=1)
    xr = x @ p['Wskip'] + p['bskip']
    beta = jax.nn.sigmoid(jnp.concatenate([out, xr, out - xr], axis=-1) @ p['Wbeta'])
    return beta * xr + (1.0 - beta) * out


def _encode(x, ei, ea, batch, params):
    x = _bn(x, params['bn_node']['g'], params['bn_node']['b'])
    ea = _bn(ea, params['bn_edge']['g'], params['bn_edge']['b'])
    x = _tconv(x, ei, ea, params['conv1'], HEADS, HID, True)
    x = jax.nn.elu(_bn(x, params['bn1']['g'], params['bn1']['b']))
    x = _tconv(x, ei, ea, params['conv2'], 1, HID, False)
    x = jax.nn.elu(_bn(x, params['bn2']['g'], params['bn2']['b']))
    cnt = jax.ops.segment_sum(jnp.ones((x.shape[0],), jnp.float32), batch, num_segments=N_GRAPHS)
    mean = jax.ops.segment_sum(x, batch, num_segments=N_GRAPHS) / jnp.maximum(cnt, 1.0)[:, None]
    mx = jax.ops.segment_max(x, batch, num_segments=N_GRAPHS)
    return jnp.concatenate([mean, mx], axis=1)


def _lstm_dir(xs, p, reverse):
    if reverse:
        xs = xs[::-1]
    bsz = xs.shape[1]
    def step(carry, xt):
        h, c = carry
        z = xt @ p['Wi'] + h @ p['Wh'] + p['b']
        i, f, g, o = jnp.split(z, 4, axis=-1)
        c = jax.nn.sigmoid(f) * c + jax.nn.sigmoid(i) * jnp.tanh(g)
        h = jax.nn.sigmoid(o) * jnp.tanh(c)
        return (h, c), h
    init = (jnp.zeros((bsz, LSTM_H), jnp.float32), jnp.zeros((bsz, LSTM_H), jnp.float32))
    _, hs = jax.lax.scan(step, init, xs)
    return hs[::-1] if reverse else hs


def _clf_body(last_ref, w1_ref, b1_ref, w2_ref, b2_ref, out_ref):
    z = last_ref[...] @ w1_ref[...] + b1_ref[...]
    hid = jnp.where(z > 0, z, jnp.exp(jnp.minimum(z, 0.0)) - 1.0)
    out_ref[...] = hid @ w2_ref[...] + b2_ref[...]


def kernel(x, edge_attr, params, edge_index, batch):
    embs = [_encode(x[t], edge_index[t], edge_attr[t], batch[t], params) for t in range(T_SNAP)]
    h = jnp.stack(embs, axis=0)
    for lp in params['lstm']:
        hf = _lstm_dir(h, lp['fwd'], False)
        hb = _lstm_dir(h, lp['bwd'], True)
        h = jnp.concatenate([hf, hb], axis=-1)
    last = h[-1]
    c = params['clf']
    return pl.pallas_call(
        _clf_body,
        out_shape=jax.ShapeDtypeStruct((last.shape[0], 2), jnp.float32),
    )(last, c['W1'], c['b1'][None, :], c['W2'], c['b2'][None, :])


# speedup vs baseline: 7.6510x; 7.6505x over previous
import functools

import jax
import jax.numpy as jnp
from jax import lax
from jax.experimental import pallas as pl
from jax.experimental.pallas import tpu as pltpu
from jax.experimental.pallas import tpu_sc as plsc

T = 4
N = 10000
E = 160000
G = 16
TN = T * N
TE = T * E
HEADS = 4
OC = 64
LH = 128

NC = 2
NS = 16
NW = NC * NS
CHUNK = 80

BN_ROWS = 1000
BE_ROWS = 2000
NBN = N // BN_ROWS
F32 = jnp.float32


def _elu(z):
    return jnp.where(z > 0, z, jnp.exp(jnp.minimum(z, 0.0)) - 1.0)


def _sigmoid(z):
    return 1.0 / (1.0 + jnp.exp(-z))


def _stats_body(x_ref, g_ref, b_ref, sc_ref, sh_ref):
    x = x_ref[0]
    m = jnp.mean(x, axis=0)
    v = jnp.mean((x - m[None, :]) ** 2, axis=0)
    sc = g_ref[0] * lax.rsqrt(v + 1e-5)
    sc_ref[0, 0] = sc
    sh_ref[0, 0] = b_ref[0] - m * sc


def _bn_stats(x_tmf, g, b):
    t, m, f = x_tmf.shape
    return pl.pallas_call(
        _stats_body,
        grid=(t,),
        in_specs=[
            pl.BlockSpec((1, m, f), lambda i: (i, 0, 0)),
            pl.BlockSpec((1, f), lambda i: (0, 0)),
            pl.BlockSpec((1, f), lambda i: (0, 0)),
        ],
        out_specs=[
            pl.BlockSpec((1, 1, f), lambda i: (i, 0, 0)),
            pl.BlockSpec((1, 1, f), lambda i: (i, 0, 0)),
        ],
        out_shape=[
            jax.ShapeDtypeStruct((t, 1, f), F32),
            jax.ShapeDtypeStruct((t, 1, f), F32),
        ],
    )(x_tmf, g.reshape(1, f), b.reshape(1, f))


def _stats16_body(x_ref, g_ref, b_ref, sc_ref, sh_ref):
    x = x_ref[0]
    s1 = jnp.sum(x, axis=0, keepdims=True)
    s2 = jnp.sum(x * x, axis=0, keepdims=True)
    r = lax.broadcasted_iota(jnp.int32, (128, 16), 0)
    c = lax.broadcasted_iota(jnp.int32, (128, 16), 1)
    sel = (r % 16 == c).astype(F32)
    m = jnp.dot(s1, sel, preferred_element_type=F32)[0] * (1.0 / E)
    sq = jnp.dot(s2, sel, preferred_element_type=F32)[0] * (1.0 / E)
    v = sq - m * m
    sc = g_ref[0] * lax.rsqrt(v + 1e-5)
    sc_ref[0, 0] = sc
    sh_ref[0, 0] = b_ref[0] - m * sc


def _bn_stats_edge(ea, g, b):
    ea_r = ea.reshape(T, E // 8, 128)
    return pl.pallas_call(
        _stats16_body,
        grid=(T,),
        in_specs=[
            pl.BlockSpec((1, E // 8, 128), lambda i: (i, 0, 0)),
            pl.BlockSpec((1, 16), lambda i: (0, 0)),
            pl.BlockSpec((1, 16), lambda i: (0, 0)),
        ],
        out_specs=[
            pl.BlockSpec((1, 1, 16), lambda i: (i, 0, 0)),
            pl.BlockSpec((1, 1, 16), lambda i: (i, 0, 0)),
        ],
        out_shape=[
            jax.ShapeDtypeStruct((T, 1, 16), F32),
            jax.ShapeDtypeStruct((T, 1, 16), F32),
        ],
    )(ea_r, g.reshape(1, 16), b.reshape(1, 16))


def _nodemm_body(do_elu, dq, dkv, x_ref, sc_ref, sh_ref, w_ref, b_ref,
                 q_ref, kv_ref, xr_ref):
    xb = x_ref[...] * sc_ref[0, 0] + sh_ref[0, 0]
    if do_elu:
        xb = _elu(xb)
    y = jnp.dot(xb, w_ref[...], preferred_element_type=F32) + b_ref[0]
    q_ref[...] = y[:, :dq]
    kv_ref[...] = y[:, dq:dq + dkv]
    xr_ref[...] = y[:, dq + dkv:]


def _node_matmuls(x_flat, scale, shift, p, do_elu):
    din = x_flat.shape[1]
    dout = p['Wq'].shape[1]
    w = jnp.concatenate([p['Wq'], p['Wk'], p['Wv'], p['Wskip']], axis=1)
    b = jnp.concatenate([p['bq'], p['bk'], p['bv'], p['bskip']]).reshape(1, -1)
    grid = TN // BN_ROWS
    return pl.pallas_call(
        functools.partial(_nodemm_body, do_elu, dout, 2 * dout),
        grid=(grid,),
        in_specs=[
            pl.BlockSpec((BN_ROWS, din), lambda j: (j, 0)),
            pl.BlockSpec((1, 1, din), lambda j: ((j * BN_ROWS) // N, 0, 0)),
            pl.BlockSpec((1, 1, din), lambda j: ((j * BN_ROWS) // N, 0, 0)),
            pl.BlockSpec((din, 4 * dout), lambda j: (0, 0)),
            pl.BlockSpec((1, 4 * dout), lambda j: (0, 0)),
        ],
        out_specs=[
            pl.BlockSpec((BN_ROWS, dout), lambda j: (j, 0)),
            pl.BlockSpec((BN_ROWS, 2 * dout), lambda j: (j, 0)),
            pl.BlockSpec((BN_ROWS, dout), lambda j: (j, 0)),
        ],
        out_shape=[
            jax.ShapeDtypeStruct((TN, dout), F32),
            jax.ShapeDtypeStruct((TN, 2 * dout), F32),
            jax.ShapeDtypeStruct((TN, dout), F32),
        ],
    )(x_flat, scale, shift, w, b)


def _edgemm_body(ea_ref, sc_ref, sh_ref, w1_ref, b1_ref, w2_ref, b2_ref,
                 e1_ref, e2_ref):
    ea = ea_ref[...] * sc_ref[0, 0] + sh_ref[0, 0]
    e1_ref[...] = jnp.dot(ea, w1_ref[...], preferred_element_type=F32) + b1_ref[0]
    e2_ref[...] = jnp.dot(ea, w2_ref[...], preferred_element_type=F32) + b2_ref[0]


def _edge_matmuls(ea_flat, scale, shift, p1, p2):
    grid = TE // BE_ROWS
    return pl.pallas_call(
        _edgemm_body,
        grid=(grid,),
        in_specs=[
            pl.BlockSpec((BE_ROWS, 16), lambda j: (j, 0)),
            pl.BlockSpec((1, 1, 16), lambda j: ((j * BE_ROWS) // E, 0, 0)),
            pl.BlockSpec((1, 1, 16), lambda j: ((j * BE_ROWS) // E, 0, 0)),
            pl.BlockSpec((16, 256), lambda j: (0, 0)),
            pl.BlockSpec((1, 256), lambda j: (0, 0)),
            pl.BlockSpec((16, 64), lambda j: (0, 0)),
            pl.BlockSpec((1, 64), lambda j: (0, 0)),
        ],
        out_specs=[
            pl.BlockSpec((BE_ROWS, 256), lambda j: (j, 0)),
            pl.BlockSpec((BE_ROWS, 64), lambda j: (j, 0)),
        ],
        out_shape=[
            jax.ShapeDtypeStruct((TE, 256), F32),
            jax.ShapeDtypeStruct((TE, 64), F32),
        ],
    )(ea_flat, scale, shift, p1['We'], p1['be'].reshape(1, -1),
      p2['We'], p2['be'].reshape(1, -1))


def _edge_att_body(heads, qg_ref, kvg_ref, e_ref, p_ref):
    d = heads * OC
    q = qg_ref[...]
    e = e_ref[...]
    kv = kvg_ref[...]
    kj = kv[:, :d] + e
    vj = kv[:, d:] + e
    bsz = q.shape[0]
    alpha = jnp.sum((q * kj).reshape(bsz, heads, OC), axis=-1) * 0.125
    ex = jnp.exp(alpha)
    evj = (ex[:, :, None] * vj.reshape(bsz, heads, OC)).reshape(bsz, d)
    wfull = p_ref.shape[2] * 2
    pad = wfull - d - heads
    row = jnp.concatenate([evj, ex, jnp.zeros((bsz, pad), F32)], axis=1)
    half = wfull // 2
    p_ref[0] = row[:, :half]
    p_ref[1] = row[:, half:]


def _edge_attention(qg, kvg, e, heads, wfull):
    d = heads * OC
    grid = TE // BE_ROWS
    return pl.pallas_call(
        functools.partial(_edge_att_body, heads),
        grid=(grid,),
        in_specs=[
            pl.BlockSpec((BE_ROWS, d), lambda j: (j, 0)),
            pl.BlockSpec((BE_ROWS, 2 * d), lambda j: (j, 0)),
            pl.BlockSpec((BE_ROWS, d), lambda j: (j, 0)),
        ],
        out_specs=pl.BlockSpec((2, BE_ROWS, wfull // 2), lambda j: (0, j, 0)),
        out_shape=jax.ShapeDtypeStruct((2, TE, wfull // 2), F32),
    )(qg, kvg, e)


def _sc_gather(table, idx):
    rows, d = table.shape
    m = idx.shape[0]
    per_w = m // NW
    n_it = per_w // CHUNK
    mesh = plsc.VectorSubcoreMesh(core_axis_name="c", subcore_axis_name="s")

    @functools.partial(
        pl.kernel,
        out_type=jax.ShapeDtypeStruct((m, d), F32),
        mesh=mesh,
        compiler_params=pltpu.CompilerParams(use_tc_tiling_on_sc=False),
        scratch_types=[
            pltpu.VMEM((CHUNK,), jnp.int32),
            pltpu.VMEM((CHUNK, d), F32),
            pltpu.SemaphoreType.DMA,
        ],
    )
    def gather_kernel(table_hbm, idx_hbm, out_hbm, idx_v, rows_v, sem):
        wid = lax.axis_index("s") * NC + lax.axis_index("c")
        base = wid * per_w

        def body(i, carry):
            off = base + i * CHUNK
            pltpu.sync_copy(idx_hbm.at[pl.ds(off, CHUNK)], idx_v)
            pltpu.async_copy(table_hbm.at[idx_v], rows_v, sem).wait()
            pltpu.sync_copy(rows_v, out_hbm.at[pl.ds(off, CHUNK)])
            return carry

        lax.fori_loop(0, n_it, body, 0)

    return gather_kernel(table, idx)


def _sc_scatter(p3, dst, zrows):
    wh = p3.shape[2]
    ep = E // NS
    n_it = ep // CHUNK
    rz = N // NS
    mesh = plsc.VectorSubcoreMesh(core_axis_name="c", subcore_axis_name="s")

    @functools.partial(
        pl.kernel,
        out_type=jax.ShapeDtypeStruct((2, TN, wh), F32),
        mesh=mesh,
        compiler_params=pltpu.CompilerParams(use_tc_tiling_on_sc=False),
        scratch_types=[
            pltpu.VMEM_SHARED((N, wh), F32),
            pltpu.VMEM((CHUNK,), jnp.int32),
            pltpu.VMEM((CHUNK, wh), F32),
        ],
    )
    def scatter_kernel(p_hbm, dst_hbm, z_hbm, out_hbm, acc_sh, idx_v, rows_v):
        c = lax.axis_index("c")
        s = lax.axis_index("s")

        def tloop(t, carry):
            pltpu.sync_copy(z_hbm, acc_sh.at[pl.ds(s * rz, rz)])
            plsc.subcore_barrier()

            def body(i, inner):
                off = t * E + s * ep + i * CHUNK
                pltpu.sync_copy(dst_hbm.at[pl.ds(off, CHUNK)], idx_v)
                pltpu.sync_copy(p_hbm.at[c, pl.ds(off, CHUNK)], rows_v)
                pltpu.sync_copy(rows_v, acc_sh.at[idx_v], add=True)
                return inner

            lax.fori_loop(0, n_it, body, 0)
            plsc.subcore_barrier()
            pltpu.sync_copy(acc_sh.at[pl.ds(s * rz, rz)],
                            out_hbm.at[c, pl.ds(t * N + s * rz, rz)])
            return carry

        lax.fori_loop(0, T, tloop, 0)

    return scatter_kernel(p3, dst, zrows)


def _asm_body(heads, acc_ref, xr_ref, wb_ref, h_ref):
    d = heads * OC
    row = jnp.concatenate([acc_ref[0], acc_ref[1]], axis=1)
    outv = row[:, :d]
    s = row[:, d:d + heads]
    bsz = outv.shape[0]
    inv = 1.0 / (s + 1e-16)
    out = (outv.reshape(bsz, heads, OC) * inv[:, :, None]).reshape(bsz, d)
    wb = wb_ref[...]
    w13 = wb[:d] + wb[2 * d:]
    w23 = wb[d:2 * d] - wb[2 * d:]
    xr = xr_ref[...]
    z = (jnp.dot(out, w13, preferred_element_type=F32)
         + jnp.dot(xr, w23, preferred_element_type=F32))
    beta = _sigmoid(z)
    h_ref[...] = beta * xr + (1.0 - beta) * out


def _assemble(acc3, xr, wbeta, heads):
    d = heads * OC
    wh = acc3.shape[2]
    grid = TN // BN_ROWS
    return pl.pallas_call(
        functools.partial(_asm_body, heads),
        grid=(grid,),
        in_specs=[
            pl.BlockSpec((2, BN_ROWS, wh), lambda j: (0, j, 0)),
            pl.BlockSpec((BN_ROWS, d), lambda j: (j, 0)),
            pl.BlockSpec((3 * d, 1), lambda j: (0, 0)),
        ],
        out_specs=pl.BlockSpec((BN_ROWS, d), lambda j: (j, 0)),
        out_shape=jax.ShapeDtypeStruct((TN, d), F32),
    )(acc3, xr, wbeta)


def _pool_body(h_ref, sc_ref, sh_ref, bf_ref, sum_ref, cnt_ref, max_ref):
    j = pl.program_id(1)
    z = h_ref[0] * sc_ref[0, 0] + sh_ref[0, 0]
    x3 = _elu(z)
    bf = bf_ref[0, 0]
    ids = lax.broadcasted_iota(jnp.int32, (1, G), 1).astype(F32)
    oh = (bf[:, None] == ids).astype(F32)
    sums = lax.dot_general(oh, x3, (((0,), (0,)), ((), ())),
                           preferred_element_type=F32)
    cnt = jnp.sum(oh, axis=0)
    parts = []
    for gi in range(G):
        mg = oh[:, gi:gi + 1] > 0.5
        parts.append(jnp.max(jnp.where(mg, x3, -jnp.inf), axis=0, keepdims=True))
    gmax = jnp.concatenate(parts, axis=0)

    @pl.when(j == 0)
    def _():
        sum_ref[0] = sums
        cnt_ref[0, 0] = cnt
        max_ref[0] = gmax

    @pl.when(j > 0)
    def _():
        sum_ref[0] += sums
        cnt_ref[0, 0] += cnt
        max_ref[0] = jnp.maximum(max_ref[0], gmax)


def _pool(h_t, scale, shift, batchf):
    return pl.pallas_call(
        _pool_body,
        grid=(T, NBN),
        in_specs=[
            pl.BlockSpec((1, BN_ROWS, OC), lambda t, j: (t, j, 0)),
            pl.BlockSpec((1, 1, OC), lambda t, j: (t, 0, 0)),
            pl.BlockSpec((1, 1, OC), lambda t, j: (t, 0, 0)),
            pl.BlockSpec((1, 1, BN_ROWS), lambda t, j: (t * NBN + j, 0, 0)),
        ],
        out_specs=[
            pl.BlockSpec((1, G, OC), lambda t, j: (t, 0, 0)),
            pl.BlockSpec((1, 1, G), lambda t, j: (t, 0, 0)),
            pl.BlockSpec((1, G, OC), lambda t, j: (t, 0, 0)),
        ],
        out_shape=[
            jax.ShapeDtypeStruct((T, G, OC), F32),
            jax.ShapeDtypeStruct((T, 1, G), F32),
            jax.ShapeDtypeStruct((T, G, OC), F32),
        ],
    )(h_t, scale, shift, batchf)


def _lstm_body(sums_ref, cnt_ref, max_ref,
               wi0f_ref, wh0f_ref, b0f_ref, wi0b_ref, wh0b_ref, b0b_ref,
               wi1f_ref, wh1f_ref, b1f_ref, wi1b_ref, wh1b_ref, b1b_ref,
               w1_ref, b1_ref, w2_ref, b2_ref, out_ref):
    cnt = jnp.maximum(cnt_ref[...], 1.0)
    xs = []
    for t in range(T):
        mean = sums_ref[t] / cnt[t, 0][:, None]
        xs.append(jnp.concatenate([mean, max_ref[t]], axis=1))

    def cell(xt, h, c, wi, wh, b):
        z = (jnp.dot(xt, wi, preferred_element_type=F32)
             + jnp.dot(h, wh, preferred_element_type=F32) + b)
        i = z[:, :LH]
        f = z[:, LH:2 * LH]
        g = z[:, 2 * LH:3 * LH]
        o = z[:, 3 * LH:]
        c2 = _sigmoid(f) * c + _sigmoid(i) * jnp.tanh(g)
        h2 = _sigmoid(o) * jnp.tanh(c2)
        return h2, c2

    for (wif, whf, bf, wib, whb, bb) in (
            (wi0f_ref, wh0f_ref, b0f_ref, wi0b_ref, wh0b_ref, b0b_ref),
            (wi1f_ref, wh1f_ref, b1f_ref, wi1b_ref, wh1b_ref, b1b_ref)):
        h = jnp.zeros((G, LH), F32)
        c = jnp.zeros((G, LH), F32)
        hf = []
        for t in range(T):
            h, c = cell(xs[t], h, c, wif[...], whf[...], bf[...])
            hf.append(h)
        h = jnp.zeros((G, LH), F32)
        c = jnp.zeros((G, LH), F32)
        hb = [None] * T
        for t in range(T - 1, -1, -1):
            h, c = cell(xs[t], h, c, wib[...], whb[...], bb[...])
            hb[t] = h
        xs = [jnp.concatenate([hf[t], hb[t]], axis=1) for t in range(T)]

    last = xs[T - 1]
    hid = _elu(jnp.dot(last, w1_ref[...], preferred_element_type=F32) + b1_ref[...])
    out_ref[...] = jnp.dot(hid, w2_ref[...], preferred_element_type=F32) + b2_ref[...]


def _lstm_head(sums, cnt, maxs, lstm_params, clf):
    l0, l1 = lstm_params
    args = [sums, cnt, maxs,
            l0['fwd']['Wi'], l0['fwd']['Wh'], l0['fwd']['b'].reshape(1, -1),
            l0['bwd']['Wi'], l0['bwd']['Wh'], l0['bwd']['b'].reshape(1, -1),
            l1['fwd']['Wi'], l1['fwd']['Wh'], l1['fwd']['b'].reshape(1, -1),
            l1['bwd']['Wi'], l1['bwd']['Wh'], l1['bwd']['b'].reshape(1, -1),
            clf['W1'], clf['b1'].reshape(1, -1), clf['W2'], clf['b2'].reshape(1, -1)]
    return pl.pallas_call(
        _lstm_body,
        out_shape=jax.ShapeDtypeStruct((G, 2), F32),
    )(*args)


def _conv_layer(x_flat, p, heads, e_flat, src_g, dst_g, dst_s, zrows, do_elu,
                scale, shift):
    d = heads * OC
    q, kv, xr = _node_matmuls(x_flat, scale, shift, p, do_elu)
    qg = _sc_gather(q, dst_g)
    kvg = _sc_gather(kv, src_g)
    wfull = {4: 288, 1: 96}[heads]
    p3 = _edge_attention(qg, kvg, e_flat, heads, wfull)
    acc3 = _sc_scatter(p3, dst_s, zrows[:, :wfull // 2])
    return _assemble(acc3, xr, p['Wbeta'], heads)


def kernel(x, edge_attr, params, edge_index, batch):
    src = edge_index[:, 0, :].astype(jnp.int32)
    dst = edge_index[:, 1, :].astype(jnp.int32)
    toff = (jnp.arange(T, dtype=jnp.int32) * N)[:, None]
    src_g = (src + toff).reshape(TE)
    dst_g = (dst + toff).reshape(TE)
    dst_s = dst.reshape(TE)
    zrows = jnp.zeros((N // NS, 144), F32)
    batchf = batch.astype(F32).reshape(T * NBN, 1, BN_ROWS)

    sc_x, sh_x = _bn_stats(x, params['bn_node']['g'], params['bn_node']['b'])
    sc_e, sh_e = _bn_stats_edge(edge_attr, params['bn_edge']['g'], params['bn_edge']['b'])
    e1, e2 = _edge_matmuls(edge_attr.reshape(TE, 16), sc_e, sh_e,
                           params['conv1'], params['conv2'])

    x_flat = x.reshape(TN, 128)
    h1 = _conv_layer(x_flat, params['conv1'], HEADS, e1, src_g, dst_g, dst_s,
                     zrows, False, sc_x, sh_x)

    sc1, sh1 = _bn_stats(h1.reshape(T, N, 256), params['bn1']['g'], params['bn1']['b'])
    h2 = _conv_layer(h1, params['conv2'], 1, e2, src_g, dst_g, dst_s,
                     zrows, True, sc1, sh1)

    sc2, sh2 = _bn_stats(h2.reshape(T, N, OC), params['bn2']['g'], params['bn2']['b'])
    sums, cnt, maxs = _pool(h2.reshape(T, N, OC), sc2, sh2, batchf)
    return _lstm_head(sums, cnt, maxs, params['lstm'], params['clf'])


# speedup vs baseline: 8.3353x; 1.0894x over previous
import functools

import jax
import jax.numpy as jnp
from jax import lax
from jax.experimental import pallas as pl
from jax.experimental.pallas import tpu as pltpu
from jax.experimental.pallas import tpu_sc as plsc

T = 4
N = 10000
E = 160000
G = 16
TN = T * N
TE = T * E
HEADS = 4
OC = 64
LH = 128

NC = 2
NS = 16
NW = NC * NS
CHUNK = 80

BN_ROWS = 1000
BE_ROWS = 2000
NBN = N // BN_ROWS
F32 = jnp.float32


def _elu(z):
    return jnp.where(z > 0, z, jnp.exp(jnp.minimum(z, 0.0)) - 1.0)


def _sigmoid(z):
    return 1.0 / (1.0 + jnp.exp(-z))


def _stats_body(x_ref, g_ref, b_ref, sc_ref, sh_ref):
    x = x_ref[0]
    m = jnp.mean(x, axis=0)
    v = jnp.mean((x - m[None, :]) ** 2, axis=0)
    sc = g_ref[0] * lax.rsqrt(v + 1e-5)
    sc_ref[0, 0] = sc
    sh_ref[0, 0] = b_ref[0] - m * sc


def _bn_stats(x_tmf, g, b):
    t, m, f = x_tmf.shape
    return pl.pallas_call(
        _stats_body,
        grid=(t,),
        in_specs=[
            pl.BlockSpec((1, m, f), lambda i: (i, 0, 0)),
            pl.BlockSpec((1, f), lambda i: (0, 0)),
            pl.BlockSpec((1, f), lambda i: (0, 0)),
        ],
        out_specs=[
            pl.BlockSpec((1, 1, f), lambda i: (i, 0, 0)),
            pl.BlockSpec((1, 1, f), lambda i: (i, 0, 0)),
        ],
        out_shape=[
            jax.ShapeDtypeStruct((t, 1, f), F32),
            jax.ShapeDtypeStruct((t, 1, f), F32),
        ],
    )(x_tmf, g.reshape(1, f), b.reshape(1, f))


def _stats16_body(x_ref, g_ref, b_ref, sc_ref, sh_ref):
    x = x_ref[0]
    s1 = jnp.sum(x, axis=0, keepdims=True)
    s2 = jnp.sum(x * x, axis=0, keepdims=True)
    r = lax.broadcasted_iota(jnp.int32, (128, 16), 0)
    c = lax.broadcasted_iota(jnp.int32, (128, 16), 1)
    sel = (r % 16 == c).astype(F32)
    m = jnp.dot(s1, sel, preferred_element_type=F32)[0] * (1.0 / E)
    sq = jnp.dot(s2, sel, preferred_element_type=F32)[0] * (1.0 / E)
    v = sq - m * m
    sc = g_ref[0] * lax.rsqrt(v + 1e-5)
    sc_ref[0, 0] = sc
    sh_ref[0, 0] = b_ref[0] - m * sc


def _bn_stats_edge(ea, g, b):
    ea_r = ea.reshape(T, E // 8, 128)
    return pl.pallas_call(
        _stats16_body,
        grid=(T,),
        in_specs=[
            pl.BlockSpec((1, E // 8, 128), lambda i: (i, 0, 0)),
            pl.BlockSpec((1, 16), lambda i: (0, 0)),
            pl.BlockSpec((1, 16), lambda i: (0, 0)),
        ],
        out_specs=[
            pl.BlockSpec((1, 1, 16), lambda i: (i, 0, 0)),
            pl.BlockSpec((1, 1, 16), lambda i: (i, 0, 0)),
        ],
        out_shape=[
            jax.ShapeDtypeStruct((T, 1, 16), F32),
            jax.ShapeDtypeStruct((T, 1, 16), F32),
        ],
    )(ea_r, g.reshape(1, 16), b.reshape(1, 16))


def _nodemm_body(do_elu, dq, dkv, x_ref, sc_ref, sh_ref, w_ref, b_ref,
                 q_ref, kv_ref, xr_ref):
    xb = x_ref[...] * sc_ref[0, 0] + sh_ref[0, 0]
    if do_elu:
        xb = _elu(xb)
    y = jnp.dot(xb, w_ref[...], preferred_element_type=F32) + b_ref[0]
    q_ref[...] = y[:, :dq]
    kv_ref[...] = y[:, dq:dq + dkv]
    xr_ref[...] = y[:, dq + dkv:]


def _node_matmuls(x_flat, scale, shift, p, do_elu):
    din = x_flat.shape[1]
    dout = p['Wq'].shape[1]
    w = jnp.concatenate([p['Wq'], p['Wk'], p['Wv'], p['Wskip']], axis=1)
    b = jnp.concatenate([p['bq'], p['bk'], p['bv'], p['bskip']]).reshape(1, -1)
    grid = TN // BN_ROWS
    return pl.pallas_call(
        functools.partial(_nodemm_body, do_elu, dout, 2 * dout),
        grid=(grid,),
        in_specs=[
            pl.BlockSpec((BN_ROWS, din), lambda j: (j, 0)),
            pl.BlockSpec((1, 1, din), lambda j: ((j * BN_ROWS) // N, 0, 0)),
            pl.BlockSpec((1, 1, din), lambda j: ((j * BN_ROWS) // N, 0, 0)),
            pl.BlockSpec((din, 4 * dout), lambda j: (0, 0)),
            pl.BlockSpec((1, 4 * dout), lambda j: (0, 0)),
        ],
        out_specs=[
            pl.BlockSpec((BN_ROWS, dout), lambda j: (j, 0)),
            pl.BlockSpec((BN_ROWS, 2 * dout), lambda j: (j, 0)),
            pl.BlockSpec((BN_ROWS, dout), lambda j: (j, 0)),
        ],
        out_shape=[
            jax.ShapeDtypeStruct((TN, dout), F32),
            jax.ShapeDtypeStruct((TN, 2 * dout), F32),
            jax.ShapeDtypeStruct((TN, dout), F32),
        ],
    )(x_flat, scale, shift, w, b)


def _edgemm_body(ea_ref, sc_ref, sh_ref, w1_ref, b1_ref, w2_ref, b2_ref,
                 e1_ref, e2_ref):
    ea = ea_ref[...] * sc_ref[0, 0] + sh_ref[0, 0]
    e1_ref[...] = jnp.dot(ea, w1_ref[...], preferred_element_type=F32) + b1_ref[0]
    e2_ref[...] = jnp.dot(ea, w2_ref[...], preferred_element_type=F32) + b2_ref[0]


def _edge_matmuls(ea_flat, scale, shift, p1, p2):
    grid = TE // BE_ROWS
    return pl.pallas_call(
        _edgemm_body,
        grid=(grid,),
        in_specs=[
            pl.BlockSpec((BE_ROWS, 16), lambda j: (j, 0)),
            pl.BlockSpec((1, 1, 16), lambda j: ((j * BE_ROWS) // E, 0, 0)),
            pl.BlockSpec((1, 1, 16), lambda j: ((j * BE_ROWS) // E, 0, 0)),
            pl.BlockSpec((16, 256), lambda j: (0, 0)),
            pl.BlockSpec((1, 256), lambda j: (0, 0)),
            pl.BlockSpec((16, 64), lambda j: (0, 0)),
            pl.BlockSpec((1, 64), lambda j: (0, 0)),
        ],
        out_specs=[
            pl.BlockSpec((BE_ROWS, 256), lambda j: (j, 0)),
            pl.BlockSpec((BE_ROWS, 64), lambda j: (j, 0)),
        ],
        out_shape=[
            jax.ShapeDtypeStruct((TE, 256), F32),
            jax.ShapeDtypeStruct((TE, 64), F32),
        ],
    )(ea_flat, scale, shift, p1['We'], p1['be'].reshape(1, -1),
      p2['We'], p2['be'].reshape(1, -1))


def _edge_att_body(heads, qg_ref, kvg_ref, e_ref, p_ref):
    d = heads * OC
    q = qg_ref[...]
    e = e_ref[...]
    kv = kvg_ref[...]
    kj = kv[:, :d] + e
    vj = kv[:, d:] + e
    bsz = q.shape[0]
    alpha = jnp.sum((q * kj).reshape(bsz, heads, OC), axis=-1) * 0.125
    ex = jnp.exp(alpha)
    evj = (ex[:, :, None] * vj.reshape(bsz, heads, OC)).reshape(bsz, d)
    wfull = p_ref.shape[2] * 2
    pad = wfull - d - heads
    row = jnp.concatenate([evj, ex, jnp.zeros((bsz, pad), F32)], axis=1)
    half = wfull // 2
    p_ref[0] = row[:, :half]
    p_ref[1] = row[:, half:]


def _edge_attention(qg, kvg, e, heads, wfull):
    d = heads * OC
    grid = TE // BE_ROWS
    return pl.pallas_call(
        functools.partial(_edge_att_body, heads),
        grid=(grid,),
        in_specs=[
            pl.BlockSpec((BE_ROWS, d), lambda j: (j, 0)),
            pl.BlockSpec((BE_ROWS, 2 * d), lambda j: (j, 0)),
            pl.BlockSpec((BE_ROWS, d), lambda j: (j, 0)),
        ],
        out_specs=pl.BlockSpec((2, BE_ROWS, wfull // 2), lambda j: (0, j, 0)),
        out_shape=jax.ShapeDtypeStruct((2, TE, wfull // 2), F32),
    )(qg, kvg, e)


def _sc_gather(table, idx):
    rows, d = table.shape
    m = idx.shape[0]
    per_w = m // NW
    n_it = per_w // CHUNK
    mesh = plsc.VectorSubcoreMesh(core_axis_name="c", subcore_axis_name="s")

    @functools.partial(
        pl.kernel,
        out_type=jax.ShapeDtypeStruct((m, d), F32),
        mesh=mesh,
        compiler_params=pltpu.CompilerParams(use_tc_tiling_on_sc=False),
        scratch_types=[
            pltpu.VMEM((per_w,), jnp.int32),
            pltpu.VMEM((CHUNK, d), F32),
            pltpu.VMEM((CHUNK, d), F32),
            pltpu.SemaphoreType.DMA,
            pltpu.SemaphoreType.DMA,
            pltpu.SemaphoreType.DMA,
            pltpu.SemaphoreType.DMA,
        ],
    )
    def gather_kernel(table_hbm, idx_hbm, out_hbm, idx_all, rows0, rows1,
                      sg0, sg1, sw0, sw1):
        wid = lax.axis_index("s") * NC + lax.axis_index("c")
        base = wid * per_w
        pltpu.sync_copy(idx_hbm.at[pl.ds(base, per_w)], idx_all)

        def src_at(i):
            return table_hbm.at[idx_all.at[pl.ds(i * CHUNK, CHUNK)]]

        def dst_at(i):
            return out_hbm.at[pl.ds(base + i * CHUNK, CHUNK)]

        pltpu.async_copy(src_at(0), rows0, sg0)

        def body(j, carry):
            i0 = 2 * j
            i1 = 2 * j + 1
            pltpu.make_async_copy(src_at(i0), rows0, sg0).wait()

            @pl.when(j > 0)
            def _():
                pltpu.make_async_copy(rows1, dst_at(i1 - 2), sw1).wait()

            pltpu.async_copy(src_at(i1), rows1, sg1)
            pltpu.async_copy(rows0, dst_at(i0), sw0)
            pltpu.make_async_copy(src_at(i1), rows1, sg1).wait()
            pltpu.make_async_copy(rows0, dst_at(i0), sw0).wait()

            @pl.when(i0 + 2 < n_it)
            def _():
                pltpu.async_copy(src_at(i0 + 2), rows0, sg0)

            pltpu.async_copy(rows1, dst_at(i1), sw1)
            return carry

        lax.fori_loop(0, n_it // 2, body, 0)
        pltpu.make_async_copy(rows1, dst_at(n_it - 1), sw1).wait()

    return gather_kernel(table, idx)


def _sc_scatter(p3, dst4, zrows):
    wh = p3.shape[2]
    ep = E // NS
    n_it = ep // CHUNK
    rz = N // NS
    mesh = plsc.VectorSubcoreMesh(core_axis_name="c", subcore_axis_name="s")

    @functools.partial(
        pl.kernel,
        out_type=jax.ShapeDtypeStruct((2, TN, wh), F32),
        mesh=mesh,
        compiler_params=pltpu.CompilerParams(use_tc_tiling_on_sc=False),
        scratch_types=[
            pltpu.VMEM_SHARED((N, wh), F32),
            pltpu.VMEM((n_it, CHUNK), jnp.int32),
            pltpu.VMEM((CHUNK, wh), F32),
            pltpu.VMEM((CHUNK, wh), F32),
            pltpu.SemaphoreType.DMA,
            pltpu.SemaphoreType.DMA,
            pltpu.SemaphoreType.DMA,
            pltpu.SemaphoreType.DMA,
        ],
    )
    def scatter_kernel(p_hbm, dst_hbm, z_hbm, out_hbm, acc_sh, idx2d,
                       rows0, rows1, sp0, sp1, sa0, sa1):
        c = lax.axis_index("c")
        s = lax.axis_index("s")

        def tloop(t, carry):
            pltpu.sync_copy(z_hbm, acc_sh.at[pl.ds(s * rz, rz)])
            pltpu.sync_copy(dst_hbm.at[t, s], idx2d)
            plsc.subcore_barrier()

            def p_src(i):
                return p_hbm.at[c, pl.ds(t * E + s * ep + i * CHUNK, CHUNK)]

            def acc_dst(i):
                return acc_sh.at[idx2d.at[i]]

            pltpu.async_copy(p_src(0), rows0, sp0)

            def body(j, cc):
                i0 = 2 * j
                i1 = 2 * j + 1
                pltpu.make_async_copy(p_src(i0), rows0, sp0).wait()

                @pl.when(j > 0)
                def _():
                    pltpu.make_async_copy(rows1, acc_dst(i1 - 2), sa1).wait()

                pltpu.async_copy(p_src(i1), rows1, sp1)
                pltpu.async_copy(rows0, acc_dst(i0), sa0, add=True)
                pltpu.make_async_copy(p_src(i1), rows1, sp1).wait()
                pltpu.make_async_copy(rows0, acc_dst(i0), sa0).wait()

                @pl.when(i0 + 2 < n_it)
                def _():
                    pltpu.async_copy(p_src(i0 + 2), rows0, sp0)

                pltpu.async_copy(rows1, acc_dst(i1), sa1, add=True)
                return cc

            lax.fori_loop(0, n_it // 2, body, 0)
            pltpu.make_async_copy(rows1, acc_dst(n_it - 2), sa1).wait()
            pltpu.make_async_copy(p_src(n_it - 1), rows0, sp0).wait()
            pltpu.sync_copy(rows0, acc_dst(n_it - 1), add=True)
            plsc.subcore_barrier()
            pltpu.sync_copy(acc_sh.at[pl.ds(s * rz, rz)],
                            out_hbm.at[c, pl.ds(t * N + s * rz, rz)])
            return carry

        lax.fori_loop(0, T, tloop, 0)

    return scatter_kernel(p3, dst4, zrows)


def _asm_body(heads, acc_ref, xr_ref, wb_ref, h_ref):
    d = heads * OC
    row = jnp.concatenate([acc_ref[0], acc_ref[1]], axis=1)
    outv = row[:, :d]
    s = row[:, d:d + heads]
    bsz = outv.shape[0]
    inv = 1.0 / (s + 1e-16)
    out = (outv.reshape(bsz, heads, OC) * inv[:, :, None]).reshape(bsz, d)
    wb = wb_ref[...]
    w13 = wb[:d] + wb[2 * d:]
    w23 = wb[d:2 * d] - wb[2 * d:]
    xr = xr_ref[...]
    z = (jnp.dot(out, w13, preferred_element_type=F32)
         + jnp.dot(xr, w23, preferred_element_type=F32))
    beta = _sigmoid(z)
    h_ref[...] = beta * xr + (1.0 - beta) * out


def _assemble(acc3, xr, wbeta, heads):
    d = heads * OC
    wh = acc3.shape[2]
    grid = TN // BN_ROWS
    return pl.pallas_call(
        functools.partial(_asm_body, heads),
        grid=(grid,),
        in_specs=[
            pl.BlockSpec((2, BN_ROWS, wh), lambda j: (0, j, 0)),
            pl.BlockSpec((BN_ROWS, d), lambda j: (j, 0)),
            pl.BlockSpec((3 * d, 1), lambda j: (0, 0)),
        ],
        out_specs=pl.BlockSpec((BN_ROWS, d), lambda j: (j, 0)),
        out_shape=jax.ShapeDtypeStruct((TN, d), F32),
    )(acc3, xr, wbeta)


def _pool_body(h_ref, sc_ref, sh_ref, bf_ref, sum_ref, cnt_ref, max_ref):
    j = pl.program_id(1)
    z = h_ref[0] * sc_ref[0, 0] + sh_ref[0, 0]
    x3 = _elu(z)
    bf = bf_ref[0, 0]
    ids = lax.broadcasted_iota(jnp.int32, (1, G), 1).astype(F32)
    oh = (bf[:, None] == ids).astype(F32)
    sums = lax.dot_general(oh, x3, (((0,), (0,)), ((), ())),
                           preferred_element_type=F32)
    cnt = jnp.sum(oh, axis=0)
    parts = []
    for gi in range(G):
        mg = oh[:, gi:gi + 1] > 0.5
        parts.append(jnp.max(jnp.where(mg, x3, -jnp.inf), axis=0, keepdims=True))
    gmax = jnp.concatenate(parts, axis=0)

    @pl.when(j == 0)
    def _():
        sum_ref[0] = sums
        cnt_ref[0, 0] = cnt
        max_ref[0] = gmax

    @pl.when(j > 0)
    def _():
        sum_ref[0] += sums
        cnt_ref[0, 0] += cnt
        max_ref[0] = jnp.maximum(max_ref[0], gmax)


def _pool(h_t, scale, shift, batchf):
    return pl.pallas_call(
        _pool_body,
        grid=(T, NBN),
        in_specs=[
            pl.BlockSpec((1, BN_ROWS, OC), lambda t, j: (t, j, 0)),
            pl.BlockSpec((1, 1, OC), lambda t, j: (t, 0, 0)),
            pl.BlockSpec((1, 1, OC), lambda t, j: (t, 0, 0)),
            pl.BlockSpec((1, 1, BN_ROWS), lambda t, j: (t * NBN + j, 0, 0)),
        ],
        out_specs=[
            pl.BlockSpec((1, G, OC), lambda t, j: (t, 0, 0)),
            pl.BlockSpec((1, 1, G), lambda t, j: (t, 0, 0)),
            pl.BlockSpec((1, G, OC), lambda t, j: (t, 0, 0)),
        ],
        out_shape=[
            jax.ShapeDtypeStruct((T, G, OC), F32),
            jax.ShapeDtypeStruct((T, 1, G), F32),
            jax.ShapeDtypeStruct((T, G, OC), F32),
        ],
    )(h_t, scale, shift, batchf)


def _lstm_body(sums_ref, cnt_ref, max_ref,
               wi0f_ref, wh0f_ref, b0f_ref, wi0b_ref, wh0b_ref, b0b_ref,
               wi1f_ref, wh1f_ref, b1f_ref, wi1b_ref, wh1b_ref, b1b_ref,
               w1_ref, b1_ref, w2_ref, b2_ref, out_ref):
    cnt = jnp.maximum(cnt_ref[...], 1.0)
    xs = []
    for t in range(T):
        mean = sums_ref[t] / cnt[t, 0][:, None]
        xs.append(jnp.concatenate([mean, max_ref[t]], axis=1))

    def cell(xt, h, c, wi, wh, b):
        z = (jnp.dot(xt, wi, preferred_element_type=F32)
             + jnp.dot(h, wh, preferred_element_type=F32) + b)
        i = z[:, :LH]
        f = z[:, LH:2 * LH]
        g = z[:, 2 * LH:3 * LH]
        o = z[:, 3 * LH:]
        c2 = _sigmoid(f) * c + _sigmoid(i) * jnp.tanh(g)
        h2 = _sigmoid(o) * jnp.tanh(c2)
        return h2, c2

    for (wif, whf, bf, wib, whb, bb) in (
            (wi0f_ref, wh0f_ref, b0f_ref, wi0b_ref, wh0b_ref, b0b_ref),
            (wi1f_ref, wh1f_ref, b1f_ref, wi1b_ref, wh1b_ref, b1b_ref)):
        h = jnp.zeros((G, LH), F32)
        c = jnp.zeros((G, LH), F32)
        hf = []
        for t in range(T):
            h, c = cell(xs[t], h, c, wif[...], whf[...], bf[...])
            hf.append(h)
        h = jnp.zeros((G, LH), F32)
        c = jnp.zeros((G, LH), F32)
        hb = [None] * T
        for t in range(T - 1, -1, -1):
            h, c = cell(xs[t], h, c, wib[...], whb[...], bb[...])
            hb[t] = h
        xs = [jnp.concatenate([hf[t], hb[t]], axis=1) for t in range(T)]

    last = xs[T - 1]
    hid = _elu(jnp.dot(last, w1_ref[...], preferred_element_type=F32) + b1_ref[...])
    out_ref[...] = jnp.dot(hid, w2_ref[...], preferred_element_type=F32) + b2_ref[...]


def _lstm_head(sums, cnt, maxs, lstm_params, clf):
    l0, l1 = lstm_params
    args = [sums, cnt, maxs,
            l0['fwd']['Wi'], l0['fwd']['Wh'], l0['fwd']['b'].reshape(1, -1),
            l0['bwd']['Wi'], l0['bwd']['Wh'], l0['bwd']['b'].reshape(1, -1),
            l1['fwd']['Wi'], l1['fwd']['Wh'], l1['fwd']['b'].reshape(1, -1),
            l1['bwd']['Wi'], l1['bwd']['Wh'], l1['bwd']['b'].reshape(1, -1),
            clf['W1'], clf['b1'].reshape(1, -1), clf['W2'], clf['b2'].reshape(1, -1)]
    return pl.pallas_call(
        _lstm_body,
        out_shape=jax.ShapeDtypeStruct((G, 2), F32),
    )(*args)


def _conv_layer(x_flat, p, heads, e_flat, src_g, dst_g, dst_s, zrows, do_elu,
                scale, shift):
    d = heads * OC
    q, kv, xr = _node_matmuls(x_flat, scale, shift, p, do_elu)
    qg = _sc_gather(q, dst_g)
    kvg = _sc_gather(kv, src_g)
    wfull = {4: 288, 1: 96}[heads]
    p3 = _edge_attention(qg, kvg, e_flat, heads, wfull)
    acc3 = _sc_scatter(p3, dst_s, zrows[:, :wfull // 2])
    return _assemble(acc3, xr, p['Wbeta'], heads)


def kernel(x, edge_attr, params, edge_index, batch):
    src = edge_index[:, 0, :].astype(jnp.int32)
    dst = edge_index[:, 1, :].astype(jnp.int32)
    toff = (jnp.arange(T, dtype=jnp.int32) * N)[:, None]
    src_g = (src + toff).reshape(TE)
    dst_g = (dst + toff).reshape(TE)
    dst4 = dst.reshape(T, NS, (E // NS) // CHUNK, CHUNK)
    zrows = jnp.zeros((N // NS, 144), F32)
    batchf = batch.astype(F32).reshape(T * NBN, 1, BN_ROWS)

    sc_x, sh_x = _bn_stats(x, params['bn_node']['g'], params['bn_node']['b'])
    sc_e, sh_e = _bn_stats_edge(edge_attr, params['bn_edge']['g'], params['bn_edge']['b'])
    e1, e2 = _edge_matmuls(edge_attr.reshape(TE, 16), sc_e, sh_e,
                           params['conv1'], params['conv2'])

    x_flat = x.reshape(TN, 128)
    h1 = _conv_layer(x_flat, params['conv1'], HEADS, e1, src_g, dst_g, dst4,
                     zrows, False, sc_x, sh_x)

    sc1, sh1 = _bn_stats(h1.reshape(T, N, 256), params['bn1']['g'], params['bn1']['b'])
    h2 = _conv_layer(h1, params['conv2'], 1, e2, src_g, dst_g, dst4,
                     zrows, True, sc1, sh1)

    sc2, sh2 = _bn_stats(h2.reshape(T, N, OC), params['bn2']['g'], params['bn2']['b'])
    sums, cnt, maxs = _pool(h2.reshape(T, N, OC), sc2, sh2, batchf)
    return _lstm_head(sums, cnt, maxs, params['lstm'], params['clf'])
